# Initial kernel scaffold; baseline (speedup 1.0000x reference)
import jax
import jax.numpy as jnp
from jax.experimental import pallas as pl


def kernel(hidden_states, W_gate):
    raise NotImplementedError("write your pallas kernel here")


# speedup vs baseline: 1.5926x; 1.5926x over previous
import functools

import jax
import jax.numpy as jnp
from jax import lax
from jax.experimental import pallas as pl
from jax.experimental.pallas import tpu as pltpu
from jax.experimental.pallas import tpu_sc as plsc

_E = 64
_K = 2
_CAPF = 1.25
_LBW = 0.01
_TBLK = 2048
_NS = 16
_NL = 16


def _tc_body(x_ref, wt_ref, probs_ref, tkp_ref, tki_ref, gsum_ref):
    i = pl.program_id(0)
    x = x_ref[...]
    wt = wt_ref[...]
    logits = jnp.dot(x, wt, preferred_element_type=jnp.float32)
    m = jnp.max(logits, axis=1, keepdims=True)
    ex = jnp.exp(logits - m)
    p = ex / jnp.sum(ex, axis=1, keepdims=True)
    probs_ref[...] = p

    iota = lax.broadcasted_iota(jnp.int32, p.shape, 1)
    m1 = jnp.max(p, axis=1, keepdims=True)
    i1 = jnp.min(jnp.where(p >= m1, iota, _E), axis=1, keepdims=True)
    pm = jnp.where(iota == i1, -1.0, p)
    m2 = jnp.max(pm, axis=1, keepdims=True)
    i2 = jnp.min(jnp.where(pm >= m2, iota, _E), axis=1, keepdims=True)
    tkp_ref[...] = jnp.concatenate([m1, m2], axis=1)
    tki_ref[...] = jnp.concatenate([i1, i2], axis=1)

    @pl.when(i == 0)
    def _():
        gsum_ref[...] = jnp.zeros_like(gsum_ref)

    gsum_ref[...] += jnp.sum(p, axis=0, keepdims=True)


def _tc_call(x, wt):
    n, d = x.shape
    t = _TBLK
    grid = n // t
    return pl.pallas_call(
        _tc_body,
        grid=(grid,),
        in_specs=[
            pl.BlockSpec((t, d), lambda i: (i, 0)),
            pl.BlockSpec((d, _E), lambda i: (0, 0)),
        ],
        out_specs=[
            pl.BlockSpec((t, _E), lambda i: (i, 0)),
            pl.BlockSpec((t, _K), lambda i: (i, 0)),
            pl.BlockSpec((t, _K), lambda i: (i, 0)),
            pl.BlockSpec((1, _E), lambda i: (0, 0)),
        ],
        out_shape=[
            jax.ShapeDtypeStruct((n, _E), jnp.float32),
            jax.ShapeDtypeStruct((n, _K), jnp.float32),
            jax.ShapeDtypeStruct((n, _K), jnp.int32),
            jax.ShapeDtypeStruct((1, _E), jnp.float32),
        ],
    )(x, wt)


def _sc_build(f_total, cap, n_tokens):
    ch = f_total // _NS
    cl = ch // _NL
    lscale = jnp.float32(_E * _LBW / (n_tokens * f_total))
    mesh = plsc.VectorSubcoreMesh(core_axis_name="c", subcore_axis_name="s")

    @functools.partial(
        pl.kernel,
        mesh=mesh,
        compiler_params=pltpu.CompilerParams(needs_layout_passes=False),
        out_type=[
            jax.ShapeDtypeStruct((f_total,), jnp.float32),
            jax.ShapeDtypeStruct((f_total,), jnp.float32),
            jax.ShapeDtypeStruct((_NL,), jnp.float32),
        ],
        scratch_types=[
            pltpu.VMEM((ch,), jnp.int32),
            pltpu.VMEM((ch,), jnp.float32),
            pltpu.VMEM((_E * _NL,), jnp.int32),
            pltpu.VMEM((_E * _NL,), jnp.int32),
            pltpu.VMEM((_E,), jnp.int32),
            pltpu.VMEM((_E,), jnp.int32),
            pltpu.VMEM((_NS * _E,), jnp.int32),
            pltpu.VMEM((ch,), jnp.float32),
            pltpu.VMEM((ch,), jnp.float32),
            pltpu.VMEM((_E,), jnp.float32),
            pltpu.VMEM((_NL,), jnp.float32),
            pltpu.VMEM_SHARED((_NS * _E,), jnp.int32),
        ],
    )
    def sc_kernel(ids_hbm, tkp_hbm, gsum_hbm, mask_hbm, norm_hbm, loss_hbm,
                  ids_v, probs_v, tbl_v, incl_v, sub_v, basew_v, allsub_v,
                  mask_v, norm_v, gsum_v, loss_v, board):
        cid = lax.axis_index("c")
        sid = lax.axis_index("s")

        @pl.when(cid == 0)
        def _body():
            base = sid * ch
            lane = lax.iota(jnp.int32, _NL)
            zero16 = jnp.zeros((_NL,), jnp.int32)

            pltpu.sync_copy(ids_hbm.at[pl.ds(base, ch)], ids_v)
            pltpu.sync_copy(tkp_hbm.at[pl.ds(base, ch)], probs_v)

            def zero_tbl(j, _):
                plsc.store_scatter(tbl_v, [j * _NL + lane], zero16)
                return 0
            lax.fori_loop(0, _E, zero_tbl, 0)

            def hist_step(i, _):
                e = plsc.load_gather(ids_v, [lane * cl + i])
                t = e * _NL + lane
                c = plsc.load_gather(tbl_v, [t])
                plsc.store_scatter(tbl_v, [t], c + 1)
                return 0
            lax.fori_loop(0, cl, hist_step, 0)

            def scan_row(e, _):
                row = plsc.load_gather(tbl_v, [e * _NL + lane])
                incl = plsc.cumsum(row)
                plsc.store_scatter(tbl_v, [e * _NL + lane], incl - row)
                plsc.store_scatter(incl_v, [e * _NL + lane], incl)
                return 0
            lax.fori_loop(0, _E, scan_row, 0)

            def grab_totals(c, _):
                er = c * _NL + lane
                s = plsc.load_gather(incl_v, [er * _NL + (_NL - 1)])
                plsc.store_scatter(sub_v, [er], s)
                return 0
            lax.fori_loop(0, _E // _NL, grab_totals, 0)

            pltpu.sync_copy(sub_v, board.at[pl.ds(sid * _E, _E)])
            plsc.subcore_barrier()
            pltpu.sync_copy(board, allsub_v)

            def zero_bw(c, _):
                plsc.store_scatter(basew_v, [c * _NL + lane], zero16)
                return 0
            lax.fori_loop(0, _E // _NL, zero_bw, 0)

            def add_prev(w, _):
                def add_chunk(c, _c):
                    er = c * _NL + lane
                    v = plsc.load_gather(allsub_v, [w * _E + er])
                    b = plsc.load_gather(basew_v, [er])
                    plsc.store_scatter(basew_v, [er], b + v)
                    return 0
                lax.fori_loop(0, _E // _NL, add_chunk, 0)
                return 0
            lax.fori_loop(0, sid, add_prev, 0)

            def fold(e, _):
                bw = plsc.load_gather(basew_v, [zero16 + e])
                row = plsc.load_gather(tbl_v, [e * _NL + lane])
                plsc.store_scatter(tbl_v, [e * _NL + lane], row + bw)
                return 0
            lax.fori_loop(0, _E, fold, 0)

            capf = jnp.float32(1.0)
            def rank_step(i, _):
                idx = lane * cl + i
                e = plsc.load_gather(ids_v, [idx])
                t = e * _NL + lane
                r = plsc.load_gather(tbl_v, [t])
                plsc.store_scatter(tbl_v, [t], r + 1)
                m = jnp.where(r < cap, capf, jnp.float32(0.0))
                plsc.store_scatter(mask_v, [idx], m)
                return 0
            lax.fori_loop(0, cl, rank_step, 0)

            def norm_step(j, _):
                i0 = j * (2 * _NL) + lane * 2
                pe = plsc.load_gather(probs_v, [i0])
                po = plsc.load_gather(probs_v, [i0 + 1])
                me = plsc.load_gather(mask_v, [i0])
                mo = plsc.load_gather(mask_v, [i0 + 1])
                a = pe * me
                b = po * mo
                s = jnp.maximum(a + b, jnp.float32(1e-8))
                plsc.store_scatter(norm_v, [i0], a / s)
                plsc.store_scatter(norm_v, [i0 + 1], b / s)
                return 0
            lax.fori_loop(0, ch // (2 * _NL), norm_step, 0)

            pltpu.sync_copy(mask_v, mask_hbm.at[pl.ds(base, ch)])
            pltpu.sync_copy(norm_v, norm_hbm.at[pl.ds(base, ch)])

            @pl.when(sid == 0)
            def _loss():
                pltpu.sync_copy(gsum_hbm, gsum_v)

                def acc_chunk(c, acc):
                    er = c * _NL + lane

                    def acc_w(w, a):
                        v = plsc.load_gather(allsub_v, [w * _E + er])
                        return a + v
                    cnt = lax.fori_loop(0, _NS, acc_w, zero16)
                    g = plsc.load_gather(gsum_v, [er])
                    return acc + g * cnt.astype(jnp.float32)

                acc = lax.fori_loop(0, _E // _NL, acc_chunk,
                                    jnp.zeros((_NL,), jnp.float32))
                loss = jnp.sum(acc) * lscale
                loss_v[...] = jnp.zeros((_NL,), jnp.float32) + loss
                pltpu.sync_copy(loss_v, loss_hbm)

    return sc_kernel


def kernel(hidden_states, W_gate):
    b, s, d = hidden_states.shape
    e = W_gate.shape[0]
    n = b * s
    f_total = n * _K
    cap = max(int((n / e) * _CAPF), 1)

    x = hidden_states.reshape(n, d)
    wt = W_gate.T
    probs, tkp, tki, gsum = _tc_call(x, wt)

    sc = _sc_build(f_total, cap, n)
    mask_f, norm_f, loss_v = sc(tki.reshape(-1), tkp.reshape(-1),
                                gsum.reshape(-1))

    return (
        tki.reshape(b, s, _K),
        norm_f.reshape(b, s, _K),
        probs.reshape(b, s, e),
        loss_v[0],
        mask_f.reshape(b, s, _K),
    )


# speedup vs baseline: 1.6062x; 1.0086x over previous
import functools

import jax
import jax.numpy as jnp
from jax import lax
from jax.experimental import pallas as pl
from jax.experimental.pallas import tpu as pltpu
from jax.experimental.pallas import tpu_sc as plsc

_E = 64
_K = 2
_CAPF = 1.25
_LBW = 0.01
_TBLK = 2048
_NS = 16
_NL = 16


def _tc_body(x_ref, wt_ref, probs_ref, tkp_ref, tki_ref, gsum_ref):
    i = pl.program_id(0)
    x = x_ref[...]
    wt = wt_ref[...]
    logits = jnp.dot(x, wt, preferred_element_type=jnp.float32)
    m = jnp.max(logits, axis=1, keepdims=True)
    ex = jnp.exp(logits - m)
    p = ex / jnp.sum(ex, axis=1, keepdims=True)
    probs_ref[...] = p

    iota = lax.broadcasted_iota(jnp.int32, p.shape, 1)
    m1 = jnp.max(p, axis=1, keepdims=True)
    i1 = jnp.min(jnp.where(p >= m1, iota, _E), axis=1, keepdims=True)
    pm = jnp.where(iota == i1, -1.0, p)
    m2 = jnp.max(pm, axis=1, keepdims=True)
    i2 = jnp.min(jnp.where(pm >= m2, iota, _E), axis=1, keepdims=True)
    tkp_ref[...] = jnp.concatenate([m1, m2], axis=1)
    tki_ref[...] = jnp.concatenate([i1, i2], axis=1)

    @pl.when(i == 0)
    def _():
        gsum_ref[...] = jnp.zeros_like(gsum_ref)

    gsum_ref[...] += jnp.sum(p, axis=0, keepdims=True)


def _tc_call(x, wt):
    n, d = x.shape
    t = _TBLK
    grid = n // t
    return pl.pallas_call(
        _tc_body,
        grid=(grid,),
        in_specs=[
            pl.BlockSpec((t, d), lambda i: (i, 0)),
            pl.BlockSpec((d, _E), lambda i: (0, 0)),
        ],
        out_specs=[
            pl.BlockSpec((t, _E), lambda i: (i, 0)),
            pl.BlockSpec((t, _K), lambda i: (i, 0)),
            pl.BlockSpec((t, _K), lambda i: (i, 0)),
            pl.BlockSpec((1, _E), lambda i: (0, 0)),
        ],
        out_shape=[
            jax.ShapeDtypeStruct((n, _E), jnp.float32),
            jax.ShapeDtypeStruct((n, _K), jnp.float32),
            jax.ShapeDtypeStruct((n, _K), jnp.int32),
            jax.ShapeDtypeStruct((1, _E), jnp.float32),
        ],
    )(x, wt)


def _sc_build(f_total, cap, n_tokens):
    ch = f_total // _NS
    cl = ch // _NL
    lscale = jnp.float32(_E * _LBW / (n_tokens * f_total))
    mesh = plsc.VectorSubcoreMesh(core_axis_name="c", subcore_axis_name="s",
                                  num_cores=1)

    @functools.partial(
        pl.kernel,
        mesh=mesh,
        compiler_params=pltpu.CompilerParams(needs_layout_passes=False),
        out_type=[
            jax.ShapeDtypeStruct((f_total,), jnp.float32),
            jax.ShapeDtypeStruct((f_total,), jnp.float32),
            jax.ShapeDtypeStruct((_NL,), jnp.float32),
        ],
        scratch_types=[
            pltpu.VMEM((ch,), jnp.int32),
            pltpu.VMEM((ch,), jnp.float32),
            pltpu.VMEM((_E * _NL,), jnp.int32),
            pltpu.VMEM((_E * _NL,), jnp.int32),
            pltpu.VMEM((_E,), jnp.int32),
            pltpu.VMEM((_E,), jnp.int32),
            pltpu.VMEM((_NS * _E,), jnp.int32),
            pltpu.VMEM((ch,), jnp.float32),
            pltpu.VMEM((ch,), jnp.float32),
            pltpu.VMEM((_E,), jnp.float32),
            pltpu.VMEM((_NL,), jnp.float32),
            pltpu.VMEM_SHARED((_NS * _E,), jnp.int32),
        ],
    )
    def sc_kernel(ids_hbm, tkp_hbm, gsum_hbm, mask_hbm, norm_hbm, loss_hbm,
                  ids_v, probs_v, tbl_v, incl_v, sub_v, basew_v, allsub_v,
                  mask_v, norm_v, gsum_v, loss_v, board):
        cid = lax.axis_index("c")
        sid = lax.axis_index("s")

        @pl.when(cid == 0)
        def _body():
            if False:
                base = sid * ch
                pltpu.sync_copy(tkp_hbm.at[pl.ds(base, ch)], probs_v)
                pltpu.sync_copy(probs_v, mask_hbm.at[pl.ds(base, ch)])
                pltpu.sync_copy(probs_v, norm_hbm.at[pl.ds(base, ch)])
                @pl.when(sid == 0)
                def _l():
                    pltpu.sync_copy(gsum_hbm.at[pl.ds(0, _NL)], loss_v)
                    pltpu.sync_copy(loss_v, loss_hbm)
                return
            base = sid * ch
            lane = lax.iota(jnp.int32, _NL)
            zero16 = jnp.zeros((_NL,), jnp.int32)

            pltpu.sync_copy(ids_hbm.at[pl.ds(base, ch)], ids_v)
            pltpu.sync_copy(tkp_hbm.at[pl.ds(base, ch)], probs_v)

            def zero_tbl(j, _):
                plsc.store_scatter(tbl_v, [j * _NL + lane], zero16)
                return 0
            lax.fori_loop(0, _E, zero_tbl, 0)

            def hist_step(i, _):
                e = plsc.load_gather(ids_v, [lane * cl + i])
                t = e * _NL + lane
                c = plsc.load_gather(tbl_v, [t])
                plsc.store_scatter(tbl_v, [t], c + 1)
                return 0
            lax.fori_loop(0, cl, hist_step, 0)

            def scan_row(e, _):
                row = plsc.load_gather(tbl_v, [e * _NL + lane])
                incl = plsc.cumsum(row)
                plsc.store_scatter(tbl_v, [e * _NL + lane], incl - row)
                plsc.store_scatter(incl_v, [e * _NL + lane], incl)
                return 0
            lax.fori_loop(0, _E, scan_row, 0)

            def grab_totals(c, _):
                er = c * _NL + lane
                s = plsc.load_gather(incl_v, [er * _NL + (_NL - 1)])
                plsc.store_scatter(sub_v, [er], s)
                return 0
            lax.fori_loop(0, _E // _NL, grab_totals, 0)

            pltpu.sync_copy(sub_v, board.at[pl.ds(sid * _E, _E)])
            plsc.subcore_barrier()
            pltpu.sync_copy(board, allsub_v)

            def zero_bw(c, _):
                plsc.store_scatter(basew_v, [c * _NL + lane], zero16)
                return 0
            lax.fori_loop(0, _E // _NL, zero_bw, 0)

            def add_prev(w, _):
                def add_chunk(c, _c):
                    er = c * _NL + lane
                    v = plsc.load_gather(allsub_v, [w * _E + er])
                    b = plsc.load_gather(basew_v, [er])
                    plsc.store_scatter(basew_v, [er], b + v)
                    return 0
                lax.fori_loop(0, _E // _NL, add_chunk, 0)
                return 0
            lax.fori_loop(0, sid, add_prev, 0)

            def fold(e, _):
                bw = plsc.load_gather(basew_v, [zero16 + e])
                row = plsc.load_gather(tbl_v, [e * _NL + lane])
                plsc.store_scatter(tbl_v, [e * _NL + lane], row + bw)
                return 0
            lax.fori_loop(0, _E, fold, 0)

            capf = jnp.float32(1.0)
            def rank_step(i, _):
                idx = lane * cl + i
                e = plsc.load_gather(ids_v, [idx])
                t = e * _NL + lane
                r = plsc.load_gather(tbl_v, [t])
                plsc.store_scatter(tbl_v, [t], r + 1)
                m = jnp.where(r < cap, capf, jnp.float32(0.0))
                plsc.store_scatter(mask_v, [idx], m)
                return 0
            lax.fori_loop(0, cl, rank_step, 0)

            def norm_step(j, _):
                i0 = j * (2 * _NL) + lane * 2
                pe = plsc.load_gather(probs_v, [i0])
                po = plsc.load_gather(probs_v, [i0 + 1])
                me = plsc.load_gather(mask_v, [i0])
                mo = plsc.load_gather(mask_v, [i0 + 1])
                a = pe * me
                b = po * mo
                s = jnp.maximum(a + b, jnp.float32(1e-8))
                plsc.store_scatter(norm_v, [i0], a / s)
                plsc.store_scatter(norm_v, [i0 + 1], b / s)
                return 0
            lax.fori_loop(0, ch // (2 * _NL), norm_step, 0)

            pltpu.sync_copy(mask_v, mask_hbm.at[pl.ds(base, ch)])
            pltpu.sync_copy(norm_v, norm_hbm.at[pl.ds(base, ch)])

            @pl.when(sid == 0)
            def _loss():
                pltpu.sync_copy(gsum_hbm, gsum_v)

                def acc_chunk(c, acc):
                    er = c * _NL + lane

                    def acc_w(w, a):
                        v = plsc.load_gather(allsub_v, [w * _E + er])
                        return a + v
                    cnt = lax.fori_loop(0, _NS, acc_w, zero16)
                    g = plsc.load_gather(gsum_v, [er])
                    return acc + g * cnt.astype(jnp.float32)

                acc = lax.fori_loop(0, _E // _NL, acc_chunk,
                                    jnp.zeros((_NL,), jnp.float32))
                loss = jnp.sum(acc) * lscale
                loss_v[...] = jnp.zeros((_NL,), jnp.float32) + loss
                pltpu.sync_copy(loss_v, loss_hbm)

    return sc_kernel


def kernel(hidden_states, W_gate):
    b, s, d = hidden_states.shape
    e = W_gate.shape[0]
    n = b * s
    f_total = n * _K
    cap = max(int((n / e) * _CAPF), 1)

    x = hidden_states.reshape(n, d)
    wt = W_gate.T
    probs, tkp, tki, gsum = _tc_call(x, wt)

    if False:
        mask_f = jnp.ones((f_total,), jnp.float32)
        norm_f = tkp.reshape(-1)
        loss_v = gsum.reshape(-1)
    else:
        sc = _sc_build(f_total, cap, n)
        mask_f, norm_f, loss_v = sc(tki.reshape(-1), tkp.reshape(-1),
                                    gsum.reshape(-1))

    return (
        tki.reshape(b, s, _K),
        norm_f.reshape(b, s, _K),
        probs.reshape(b, s, e),
        loss_v[0],
        mask_f.reshape(b, s, _K),
    )


# speedup vs baseline: 2.1196x; 1.3196x over previous
import functools

import jax
import jax.numpy as jnp
from jax import lax
from jax.experimental import pallas as pl
from jax.experimental.pallas import tpu as pltpu
from jax.experimental.pallas import tpu_sc as plsc

_E = 64
_K = 2
_CAPF = 1.25
_LBW = 0.01
_TBLK = 2048
_NS = 16
_NL = 16


def _tc1_body(x_ref, w_ref, probs_ref, pack_ref, tkp0_ref, tkp1_ref,
              gsum_ref):
    i = pl.program_id(0)
    t = x_ref.shape[0]
    x = x_ref[...]
    w = w_ref[...]
    nt = (((1,), (1,)), ((), ()))

    logits = lax.dot_general(x, w, nt, preferred_element_type=jnp.float32)
    m = jnp.max(logits, axis=1, keepdims=True)
    ex = jnp.exp(logits - m)
    p = ex / jnp.sum(ex, axis=1, keepdims=True)
    probs_ref[...] = p

    lt = lax.dot_general(w, x, nt, preferred_element_type=jnp.float32)
    mt = jnp.max(lt, axis=0, keepdims=True)
    et = jnp.exp(lt - mt)
    st = jnp.sum(et, axis=0, keepdims=True)
    pt = et / st
    iota = lax.broadcasted_iota(jnp.int32, pt.shape, 0)
    m1 = jnp.max(pt, axis=0, keepdims=True)
    i1 = jnp.min(jnp.where(pt >= m1, iota, _E), axis=0, keepdims=True)
    pm = jnp.where(iota == i1, -1.0, pt)
    m2 = jnp.max(pm, axis=0, keepdims=True)
    i2 = jnp.min(jnp.where(pm >= m2, iota, _E), axis=0, keepdims=True)

    pack_ref[...] = (i1 * _E + i2).reshape(t)
    tkp0_ref[...] = m1.reshape(t)
    tkp1_ref[...] = m2.reshape(t)

    @pl.when(i == 0)
    def _():
        gsum_ref[...] = jnp.zeros_like(gsum_ref)

    gsum_ref[0:1, 0:_E] += jnp.sum(p, axis=0, keepdims=True)


def _tc1_call(x, w):
    n, d = x.shape
    t = _TBLK
    grid = n // t
    return pl.pallas_call(
        _tc1_body,
        grid=(grid,),
        in_specs=[
            pl.BlockSpec((t, d), lambda i: (i, 0)),
            pl.BlockSpec((_E, d), lambda i: (0, 0)),
        ],
        out_specs=[
            pl.BlockSpec((t, _E), lambda i: (i, 0)),
            pl.BlockSpec((t,), lambda i: (i,)),
            pl.BlockSpec((t,), lambda i: (i,)),
            pl.BlockSpec((t,), lambda i: (i,)),
            pl.BlockSpec((8, 128), lambda i: (0, 0)),
        ],
        out_shape=[
            jax.ShapeDtypeStruct((n, _E), jnp.float32),
            jax.ShapeDtypeStruct((n,), jnp.int32),
            jax.ShapeDtypeStruct((n,), jnp.float32),
            jax.ShapeDtypeStruct((n,), jnp.float32),
            jax.ShapeDtypeStruct((8, 128), jnp.float32),
        ],
    )(x, w)


def _sc_build(n_tokens, cap):
    ch = n_tokens // _NS
    cl = ch // _NL
    f_total = n_tokens * _K
    lscale = jnp.float32(_E * _LBW / (n_tokens * f_total))
    mesh = plsc.VectorSubcoreMesh(core_axis_name="c", subcore_axis_name="s",
                                  num_cores=1)

    @functools.partial(
        pl.kernel,
        mesh=mesh,
        compiler_params=pltpu.CompilerParams(needs_layout_passes=False),
        out_type=[
            jax.ShapeDtypeStruct((n_tokens,), jnp.int32),
            jax.ShapeDtypeStruct((_NL,), jnp.float32),
        ],
        scratch_types=[
            pltpu.VMEM((ch,), jnp.int32),
            pltpu.VMEM((_E * _NL,), jnp.int32),
            pltpu.VMEM((_E * _NL,), jnp.int32),
            pltpu.VMEM((_E,), jnp.int32),
            pltpu.VMEM((_E,), jnp.int32),
            pltpu.VMEM((_NS * _E,), jnp.int32),
            pltpu.VMEM((ch,), jnp.int32),
            pltpu.VMEM((128,), jnp.float32),
            pltpu.VMEM((_NL,), jnp.float32),
            pltpu.VMEM_SHARED((_NS * _E,), jnp.int32),
        ],
    )
    def sc_kernel(pack_hbm, gsum_hbm, maskp_hbm, loss_hbm,
                  pack_v, tbl_v, incl_v, sub_v, basew_v, allsub_v,
                  maskp_v, gsum_v, loss_v, board):
        sid = lax.axis_index("s")
        base = sid * ch
        lane = lax.iota(jnp.int32, _NL)
        zero16 = jnp.zeros((_NL,), jnp.int32)

        pltpu.sync_copy(pack_hbm.at[pl.ds(base, ch)], pack_v)

        def zero_tbl(j, _):
            plsc.store_scatter(tbl_v, [j * _NL + lane], zero16)
            return 0
        lax.fori_loop(0, _E, zero_tbl, 0)

        def hist_step(j, _):
            v = plsc.load_gather(pack_v, [lane * cl + j])
            e0 = v >> 6
            e1 = v & 63
            c0 = plsc.load_gather(tbl_v, [e0 * _NL + lane])
            plsc.store_scatter(tbl_v, [e0 * _NL + lane], c0 + 1)
            c1 = plsc.load_gather(tbl_v, [e1 * _NL + lane])
            plsc.store_scatter(tbl_v, [e1 * _NL + lane], c1 + 1)
            return 0
        lax.fori_loop(0, cl, hist_step, 0)

        def scan_row(e, _):
            row = plsc.load_gather(tbl_v, [e * _NL + lane])
            incl = plsc.cumsum(row)
            plsc.store_scatter(tbl_v, [e * _NL + lane], incl - row)
            plsc.store_scatter(incl_v, [e * _NL + lane], incl)
            return 0
        lax.fori_loop(0, _E, scan_row, 0)

        def grab_totals(c, _):
            er = c * _NL + lane
            s = plsc.load_gather(incl_v, [er * _NL + (_NL - 1)])
            plsc.store_scatter(sub_v, [er], s)
            return 0
        lax.fori_loop(0, _E // _NL, grab_totals, 0)

        pltpu.sync_copy(sub_v, board.at[pl.ds(sid * _E, _E)])
        plsc.subcore_barrier()
        pltpu.sync_copy(board, allsub_v)

        def zero_bw(c, _):
            plsc.store_scatter(basew_v, [c * _NL + lane], zero16)
            return 0
        lax.fori_loop(0, _E // _NL, zero_bw, 0)

        def add_prev(w, _):
            def add_chunk(c, _c):
                er = c * _NL + lane
                v = plsc.load_gather(allsub_v, [w * _E + er])
                b = plsc.load_gather(basew_v, [er])
                plsc.store_scatter(basew_v, [er], b + v)
                return 0
            lax.fori_loop(0, _E // _NL, add_chunk, 0)
            return 0
        lax.fori_loop(0, sid, add_prev, 0)

        def fold(e, _):
            bw = plsc.load_gather(basew_v, [zero16 + e])
            row = plsc.load_gather(tbl_v, [e * _NL + lane])
            plsc.store_scatter(tbl_v, [e * _NL + lane], row + bw)
            return 0
        lax.fori_loop(0, _E, fold, 0)

        one = jnp.full((_NL,), 1, jnp.int32)
        zero = zero16
        def rank_step(j, _):
            idx = lane * cl + j
            v = plsc.load_gather(pack_v, [idx])
            e0 = v >> 6
            e1 = v & 63
            r0 = plsc.load_gather(tbl_v, [e0 * _NL + lane])
            plsc.store_scatter(tbl_v, [e0 * _NL + lane], r0 + 1)
            r1 = plsc.load_gather(tbl_v, [e1 * _NL + lane])
            plsc.store_scatter(tbl_v, [e1 * _NL + lane], r1 + 1)
            mp = jnp.where(r0 < cap, one, zero) + \
                 jnp.where(r1 < cap, one + one, zero)
            plsc.store_scatter(maskp_v, [idx], mp)
            return 0
        lax.fori_loop(0, cl, rank_step, 0)

        pltpu.sync_copy(maskp_v, maskp_hbm.at[pl.ds(base, ch)])

        @pl.when(sid == 0)
        def _loss():
            pltpu.sync_copy(gsum_hbm.at[0], gsum_v)

            def acc_chunk(c, acc):
                er = c * _NL + lane

                def acc_w(w, a):
                    v = plsc.load_gather(allsub_v, [w * _E + er])
                    return a + v
                cnt = lax.fori_loop(0, _NS, acc_w, zero16)
                g = plsc.load_gather(gsum_v, [er])
                return acc + g * cnt.astype(jnp.float32)

            acc = lax.fori_loop(0, _E // _NL, acc_chunk,
                                jnp.zeros((_NL,), jnp.float32))
            loss = jnp.sum(acc) * lscale
            loss_v[...] = jnp.zeros((_NL,), jnp.float32) + loss
            pltpu.sync_copy(loss_v, loss_hbm)

    return sc_kernel


def _tc2_body(pack_ref, tkp0_ref, tkp1_ref, maskp_ref,
              tki_ref, norm_ref, maskf_ref):
    rows = pack_ref.shape[0]
    v = pack_ref[...]
    mp = maskp_ref[...]
    p0 = tkp0_ref[...]
    p1 = tkp1_ref[...]
    i1 = v >> 6
    i2 = v & 63
    m0 = (mp & 1).astype(jnp.float32)
    m1 = ((mp >> 1) & 1).astype(jnp.float32)
    a = p0 * m0
    b = p1 * m1
    s = jnp.maximum(a + b, 1e-8)
    n0 = a / s
    n1 = b / s

    tr = lambda z: lax.transpose(z, (1, 0))
    ti1 = tr(i1)
    ti2 = tr(i2)
    tn0 = tr(n0)
    tn1 = tr(n1)
    tm0 = tr(m0)
    tm1 = tr(m1)
    for r in range(rows):
        sl = pl.ds(r * 128, 128)
        tki_ref[sl, :] = jnp.concatenate(
            [ti1[:, r:r + 1], ti2[:, r:r + 1]], axis=1)
        norm_ref[sl, :] = jnp.concatenate(
            [tn0[:, r:r + 1], tn1[:, r:r + 1]], axis=1)
        maskf_ref[sl, :] = jnp.concatenate(
            [tm0[:, r:r + 1], tm1[:, r:r + 1]], axis=1)


def _tc2_call(pack2, tkp02, tkp12, maskp2):
    nr = pack2.shape[0]
    rows = _TBLK // 128
    grid = nr // rows
    n = nr * 128
    return pl.pallas_call(
        _tc2_body,
        grid=(grid,),
        in_specs=[pl.BlockSpec((rows, 128), lambda i: (i, 0))] * 4,
        out_specs=[pl.BlockSpec((_TBLK, _K), lambda i: (i, 0))] * 3,
        out_shape=[
            jax.ShapeDtypeStruct((n, _K), jnp.int32),
            jax.ShapeDtypeStruct((n, _K), jnp.float32),
            jax.ShapeDtypeStruct((n, _K), jnp.float32),
        ],
    )(pack2, tkp02, tkp12, maskp2)


def kernel(hidden_states, W_gate):
    b, s, d = hidden_states.shape
    e = W_gate.shape[0]
    n = b * s
    cap = max(int((n / e) * _CAPF), 1)

    x = hidden_states.reshape(n, d)
    probs, pack, tkp0, tkp1, gsum = _tc1_call(x, W_gate)

    sc = _sc_build(n, cap)
    maskp, loss_v = sc(pack, gsum)

    q = n // 128
    tki, norm, maskf = _tc2_call(
        pack.reshape(q, 128), tkp0.reshape(q, 128),
        tkp1.reshape(q, 128), maskp.reshape(q, 128))

    return (
        tki.reshape(b, s, _K),
        norm.reshape(b, s, _K),
        probs.reshape(b, s, e),
        loss_v[0],
        maskf.reshape(b, s, _K),
    )
